# Initial kernel scaffold; baseline (speedup 1.0000x reference)
import jax
import jax.numpy as jnp
from jax.experimental import pallas as pl


def kernel(x, edge_index, W_l, b_l, W_r, b_r):
    raise NotImplementedError("write your pallas kernel here")


# speedup vs baseline: 2.5108x; 2.5108x over previous
import functools

import jax
import jax.numpy as jnp
from jax import lax
from jax.experimental import pallas as pl
from jax.experimental.pallas import tpu as pltpu
from jax.experimental.pallas import tpu_sc as plsc

N = 10000
E = 160000
D = 256
DH = 128
NC = 2
NS = 16
K = 128
CH = 80
EP = NS * CH * K
RPS = 632
NPAD = NS * RPS

_mesh = plsc.VectorSubcoreMesh(core_axis_name="c", subcore_axis_name="s")


@functools.partial(
    pl.kernel,
    mesh=_mesh,
    compiler_params=pltpu.CompilerParams(use_tc_tiling_on_sc=False),
    out_type=[
        jax.ShapeDtypeStruct((NC, NPAD, DH), jnp.float32),
        jax.ShapeDtypeStruct((NPAD, 16), jnp.float32),
    ],
    scratch_types=[
        pltpu.VMEM((K,), jnp.int32),
        pltpu.VMEM((K,), jnp.int32),
        pltpu.VMEM((K, DH), jnp.float32),
        pltpu.VMEM((K, 16), jnp.float32),
        pltpu.VMEM_SHARED((NPAD, DH), jnp.float32),
        pltpu.VMEM_SHARED((NPAD, 16), jnp.float32),
        pltpu.SemaphoreType.DMA,
    ],
)
def _sc_aggregate(xcat, rowp, colp, zacc, zcnt, ones, out, cntout,
                  idxr, idxc, gbuf, obuf, acc, cnt, sem):
    c = lax.axis_index("c")
    s = lax.axis_index("s")
    base = s * RPS

    pltpu.sync_copy(zacc, acc.at[pl.ds(base, RPS)])
    pltpu.sync_copy(zcnt, cnt.at[pl.ds(base, RPS)])
    pltpu.sync_copy(ones, obuf)
    plsc.subcore_barrier()

    @pl.loop(0, CH)
    def _(g):
        pltpu.sync_copy(rowp.at[pl.ds(((c * NS + s) * CH + g) * K, K)], idxr)
        pltpu.sync_copy(colp.at[pl.ds((s * CH + g) * K, K)], idxc)
        pltpu.async_copy(xcat.at[idxr], gbuf, sem).wait()
        pltpu.sync_copy(gbuf, acc.at[idxc], add=True)
        pltpu.sync_copy(obuf, cnt.at[idxc], add=True)

    plsc.subcore_barrier()

    pltpu.sync_copy(acc.at[pl.ds(base, RPS)], out.at[c, pl.ds(base, RPS)])

    @pl.when(c == 0)
    def _():
        pltpu.sync_copy(cnt.at[pl.ds(base, RPS)], cntout.at[pl.ds(base, RPS)])


def _tc_body(sum_ref, x_ref, c_ref, wl_ref, wr_ref, b_ref, o_ref):
    rec = 1.0 / jnp.maximum(c_ref[...][:, 0:1], 1.0)
    outl = jnp.dot(sum_ref[...], wl_ref[...],
                   preferred_element_type=jnp.float32,
                   precision=lax.Precision.HIGHEST)
    outr = jnp.dot(x_ref[...], wr_ref[...],
                   preferred_element_type=jnp.float32,
                   precision=lax.Precision.HIGHEST)
    o_ref[...] = outl * rec + outr + b_ref[...]


def _tc_linear(summed, x, cnt, wlT, wrT, bias):
    blk = 1000
    return pl.pallas_call(
        _tc_body,
        grid=(N // blk,),
        in_specs=[
            pl.BlockSpec((blk, D), lambda i: (i, 0)),
            pl.BlockSpec((blk, D), lambda i: (i, 0)),
            pl.BlockSpec((blk, 16), lambda i: (i, 0)),
            pl.BlockSpec((D, D), lambda i: (0, 0)),
            pl.BlockSpec((D, D), lambda i: (0, 0)),
            pl.BlockSpec((1, D), lambda i: (0, 0)),
        ],
        out_specs=pl.BlockSpec((blk, D), lambda i: (i, 0)),
        out_shape=jax.ShapeDtypeStruct((N, D), jnp.float32),
    )(summed, x, cnt, wlT, wrT, bias)


@jax.jit
def kernel(x, edge_index, W_l, b_l, W_r, b_r):
    row = edge_index[0].astype(jnp.int32)
    col = edge_index[1].astype(jnp.int32)

    xcat = jnp.concatenate([x[:, :DH], x[:, DH:]], axis=0)

    pad = EP - E
    rowp = jnp.pad(row, (0, pad))
    colp = jnp.pad(col, (0, pad), constant_values=N)
    rowp2 = jnp.stack([rowp, rowp + N]).reshape(NC * EP)
    colp2 = colp.reshape(EP)

    zacc = jnp.zeros((RPS, DH), jnp.float32)
    zcnt = jnp.zeros((RPS, 16), jnp.float32)
    ones = jnp.ones((K, 16), jnp.float32)

    summ2, cnt = _sc_aggregate(xcat, rowp2, colp2, zacc, zcnt, ones)
    summed = jnp.concatenate([summ2[0, :N], summ2[1, :N]], axis=1)

    return _tc_linear(summed, x, cnt[:N], W_l.T, W_r.T,
                      (b_l + b_r).reshape(1, D))


# speedup vs baseline: 3.1711x; 1.2630x over previous
import functools

import jax
import jax.numpy as jnp
from jax import lax
from jax.experimental import pallas as pl
from jax.experimental.pallas import tpu as pltpu
from jax.experimental.pallas import tpu_sc as plsc

N = 10000
E = 160000
D = 256
DH = 128
NC = 2
NS = 16
K = 128
CH = 80
EP = NS * CH * K
RPS = 632
NPAD = NS * RPS

_mesh = plsc.VectorSubcoreMesh(core_axis_name="c", subcore_axis_name="s")


@functools.partial(
    pl.kernel,
    mesh=_mesh,
    compiler_params=pltpu.CompilerParams(use_tc_tiling_on_sc=False),
    out_type=[
        jax.ShapeDtypeStruct((NC, NPAD, DH), jnp.float32),
        jax.ShapeDtypeStruct((NPAD, 16), jnp.float32),
    ],
    scratch_types=[
        pltpu.VMEM((K,), jnp.int32),
        pltpu.VMEM((K,), jnp.int32),
        pltpu.VMEM((K,), jnp.int32),
        pltpu.VMEM((K,), jnp.int32),
        pltpu.VMEM((K, DH), jnp.float32),
        pltpu.VMEM((K, DH), jnp.float32),
        pltpu.VMEM((K, 16), jnp.float32),
        pltpu.VMEM_SHARED((NPAD, DH), jnp.float32),
        pltpu.VMEM_SHARED((NPAD, 16), jnp.float32),
        pltpu.SemaphoreType.DMA,
        pltpu.SemaphoreType.DMA,
    ],
)
def _sc_aggregate(xcat, rowp, colp, zacc, zcnt, ones, out, cntout,
                  idxr0, idxc0, idxr1, idxc1, gbuf0, gbuf1, obuf,
                  acc, cnt, sem0, sem1):
    c = lax.axis_index("c")
    s = lax.axis_index("s")
    base = s * RPS
    ebase = (c * NS + s) * CH
    cbase = s * CH

    pltpu.sync_copy(zacc, acc.at[pl.ds(base, RPS)])
    pltpu.sync_copy(zcnt, cnt.at[pl.ds(base, RPS)])
    pltpu.sync_copy(ones, obuf)
    plsc.subcore_barrier()

    def load_idx(g, ir, ic):
        pltpu.sync_copy(rowp.at[pl.ds((ebase + g) * K, K)], ir)
        pltpu.sync_copy(colp.at[pl.ds((cbase + g) * K, K)], ic)

    def fire(ir, gb, sem):
        pltpu.async_copy(xcat.at[ir], gb, sem)

    def drain(ir, gb, ic, sem):
        pltpu.make_async_copy(xcat.at[ir], gb, sem).wait()
        pltpu.sync_copy(gb, acc.at[ic], add=True)
        pltpu.sync_copy(obuf, cnt.at[ic], add=True)

    load_idx(0, idxr0, idxc0)
    fire(idxr0, gbuf0, sem0)

    @pl.loop(0, CH, step=2)
    def _(g):
        load_idx(g + 1, idxr1, idxc1)
        fire(idxr1, gbuf1, sem1)
        drain(idxr0, gbuf0, idxc0, sem0)

        @pl.when(g + 2 < CH)
        def _():
            load_idx(g + 2, idxr0, idxc0)
            fire(idxr0, gbuf0, sem0)

        drain(idxr1, gbuf1, idxc1, sem1)

    plsc.subcore_barrier()

    pltpu.sync_copy(acc.at[pl.ds(base, RPS)], out.at[c, pl.ds(base, RPS)])

    @pl.when(c == 0)
    def _():
        pltpu.sync_copy(cnt.at[pl.ds(base, RPS)], cntout.at[pl.ds(base, RPS)])


def _tc_body(sum_ref, x_ref, c_ref, wl_ref, wr_ref, b_ref, o_ref):
    rec = 1.0 / jnp.maximum(c_ref[...][:, 0:1], 1.0)
    outl = jnp.dot(sum_ref[...], wl_ref[...],
                   preferred_element_type=jnp.float32,
                   precision=lax.Precision.HIGHEST)
    outr = jnp.dot(x_ref[...], wr_ref[...],
                   preferred_element_type=jnp.float32,
                   precision=lax.Precision.HIGHEST)
    o_ref[...] = outl * rec + outr + b_ref[...]


def _tc_linear(summed, x, cnt, wlT, wrT, bias):
    blk = 1000
    return pl.pallas_call(
        _tc_body,
        grid=(N // blk,),
        in_specs=[
            pl.BlockSpec((blk, D), lambda i: (i, 0)),
            pl.BlockSpec((blk, D), lambda i: (i, 0)),
            pl.BlockSpec((blk, 16), lambda i: (i, 0)),
            pl.BlockSpec((D, D), lambda i: (0, 0)),
            pl.BlockSpec((D, D), lambda i: (0, 0)),
            pl.BlockSpec((1, D), lambda i: (0, 0)),
        ],
        out_specs=pl.BlockSpec((blk, D), lambda i: (i, 0)),
        out_shape=jax.ShapeDtypeStruct((N, D), jnp.float32),
    )(summed, x, cnt, wlT, wrT, bias)


@jax.jit
def kernel(x, edge_index, W_l, b_l, W_r, b_r):
    row = edge_index[0].astype(jnp.int32)
    col = edge_index[1].astype(jnp.int32)

    xcat = jnp.concatenate([x[:, :DH], x[:, DH:]], axis=0)

    pad = EP - E
    rowp = jnp.pad(row, (0, pad))
    colp = jnp.pad(col, (0, pad), constant_values=N)
    rowp2 = jnp.stack([rowp, rowp + N]).reshape(NC * EP)
    colp2 = colp.reshape(EP)

    zacc = jnp.zeros((RPS, DH), jnp.float32)
    zcnt = jnp.zeros((RPS, 16), jnp.float32)
    ones = jnp.ones((K, 16), jnp.float32)

    summ2, cnt = _sc_aggregate(xcat, rowp2, colp2, zacc, zcnt, ones)
    summed = jnp.concatenate([summ2[0, :N], summ2[1, :N]], axis=1)

    return _tc_linear(summed, x, cnt[:N], W_l.T, W_r.T,
                      (b_l + b_r).reshape(1, D))


# speedup vs baseline: 3.3478x; 1.0557x over previous
import functools

import jax
import jax.numpy as jnp
from jax import lax
from jax.experimental import pallas as pl
from jax.experimental.pallas import tpu as pltpu
from jax.experimental.pallas import tpu_sc as plsc

N = 10000
E = 160000
D = 256
DH = 128
NC = 2
NS = 16
K = 128
CH = 80
SEG = 16
EP = NS * CH * K
RPS = 632
NPAD = NS * RPS

_mesh = plsc.VectorSubcoreMesh(core_axis_name="c", subcore_axis_name="s")


@functools.partial(
    pl.kernel,
    mesh=_mesh,
    compiler_params=pltpu.CompilerParams(use_tc_tiling_on_sc=False),
    out_type=[
        jax.ShapeDtypeStruct((NC, NPAD, DH), jnp.float32),
        jax.ShapeDtypeStruct((NPAD, 16), jnp.float32),
    ],
    scratch_types=[
        pltpu.VMEM((SEG, K), jnp.int32),
        pltpu.VMEM((SEG, K), jnp.int32),
        pltpu.VMEM((K, DH), jnp.float32),
        pltpu.VMEM((K, DH), jnp.float32),
        pltpu.VMEM((K, 16), jnp.float32),
        pltpu.VMEM_SHARED((NPAD, DH), jnp.float32),
        pltpu.VMEM_SHARED((NPAD, 16), jnp.float32),
        pltpu.SemaphoreType.DMA,
        pltpu.SemaphoreType.DMA,
        pltpu.SemaphoreType.DMA,
        pltpu.SemaphoreType.DMA,
        pltpu.SemaphoreType.DMA,
        pltpu.SemaphoreType.DMA,
    ],
)
def _sc_aggregate(xcat, rowp, colp, zacc, zcnt, ones, out, cntout,
                  rbuf, cbuf, gbuf0, gbuf1, obuf, acc, cnt,
                  gsem0, gsem1, asem0, asem1, csem0, csem1):
    c = lax.axis_index("c")
    s = lax.axis_index("s")
    base = s * RPS
    ebase = (c * NS + s) * CH
    cbase = s * CH

    pltpu.sync_copy(zacc, acc.at[pl.ds(base, RPS)])
    pltpu.sync_copy(zcnt, cnt.at[pl.ds(base, RPS)])
    pltpu.sync_copy(ones, obuf)
    plsc.subcore_barrier()

    def fire_gather(j, gb, sem):
        pltpu.async_copy(xcat.at[rbuf.at[j]], gb, sem)

    def wait_gather(gb, sem):
        pltpu.make_async_copy(xcat.at[rbuf.at[0]], gb, sem).wait()

    def fire_scatter(j, gb, asem, csem):
        pltpu.async_copy(gb, acc.at[cbuf.at[j]], asem, add=True)
        pltpu.async_copy(obuf, cnt.at[cbuf.at[j]], csem, add=True)

    def wait_scatter(gb, asem, csem):
        pltpu.make_async_copy(gb, acc.at[cbuf.at[0]], asem).wait()
        pltpu.make_async_copy(obuf, cnt.at[cbuf.at[0]], csem).wait()

    @pl.loop(0, CH, step=SEG)
    def _(g0):
        pltpu.sync_copy(rowp.at[pl.ds(ebase + g0, SEG)], rbuf)
        pltpu.sync_copy(colp.at[pl.ds(cbase + g0, SEG)], cbuf)
        fire_gather(0, gbuf0, gsem0)
        fire_gather(1, gbuf1, gsem1)

        @pl.loop(0, SEG, step=2)
        def _(j):
            wait_gather(gbuf0, gsem0)
            fire_scatter(j, gbuf0, asem0, csem0)
            wait_gather(gbuf1, gsem1)
            fire_scatter(j + 1, gbuf1, asem1, csem1)
            wait_scatter(gbuf0, asem0, csem0)

            @pl.when(j + 2 < SEG)
            def _():
                fire_gather(j + 2, gbuf0, gsem0)

            wait_scatter(gbuf1, asem1, csem1)

            @pl.when(j + 3 < SEG)
            def _():
                fire_gather(j + 3, gbuf1, gsem1)

    plsc.subcore_barrier()

    pltpu.sync_copy(acc.at[pl.ds(base, RPS)], out.at[c, pl.ds(base, RPS)])

    @pl.when(c == 0)
    def _():
        pltpu.sync_copy(cnt.at[pl.ds(base, RPS)], cntout.at[pl.ds(base, RPS)])


def _tc_body(sum_ref, x_ref, c_ref, wl_ref, wr_ref, b_ref, o_ref):
    rec = 1.0 / jnp.maximum(c_ref[...][:, 0:1], 1.0)
    outl = jnp.dot(sum_ref[...], wl_ref[...],
                   preferred_element_type=jnp.float32,
                   precision=lax.Precision.HIGHEST)
    outr = jnp.dot(x_ref[...], wr_ref[...],
                   preferred_element_type=jnp.float32,
                   precision=lax.Precision.HIGHEST)
    o_ref[...] = outl * rec + outr + b_ref[...]


def _tc_linear(summed, x, cnt, wlT, wrT, bias):
    blk = 1000
    return pl.pallas_call(
        _tc_body,
        grid=(N // blk,),
        in_specs=[
            pl.BlockSpec((blk, D), lambda i: (i, 0)),
            pl.BlockSpec((blk, D), lambda i: (i, 0)),
            pl.BlockSpec((blk, 16), lambda i: (i, 0)),
            pl.BlockSpec((D, D), lambda i: (0, 0)),
            pl.BlockSpec((D, D), lambda i: (0, 0)),
            pl.BlockSpec((1, D), lambda i: (0, 0)),
        ],
        out_specs=pl.BlockSpec((blk, D), lambda i: (i, 0)),
        out_shape=jax.ShapeDtypeStruct((N, D), jnp.float32),
    )(summed, x, cnt, wlT, wrT, bias)


@jax.jit
def kernel(x, edge_index, W_l, b_l, W_r, b_r):
    row = edge_index[0].astype(jnp.int32)
    col = edge_index[1].astype(jnp.int32)

    xcat = jnp.concatenate([x[:, :DH], x[:, DH:]], axis=0)

    pad = EP - E
    rowp = jnp.pad(row, (0, pad))
    colp = jnp.pad(col, (0, pad), constant_values=N)
    rowp2 = jnp.stack([rowp, rowp + N]).reshape(NC * NS * CH, K)
    colp2 = colp.reshape(NS * CH, K)

    zacc = jnp.zeros((RPS, DH), jnp.float32)
    zcnt = jnp.zeros((RPS, 16), jnp.float32)
    ones = jnp.ones((K, 16), jnp.float32)

    summ2, cnt = _sc_aggregate(xcat, rowp2, colp2, zacc, zcnt, ones)
    summed = jnp.concatenate([summ2[0, :N], summ2[1, :N]], axis=1)

    return _tc_linear(summed, x, cnt[:N], W_l.T, W_r.T,
                      (b_l + b_r).reshape(1, D))
